# Initial kernel scaffold; baseline (speedup 1.0000x reference)
import jax
import jax.numpy as jnp
from jax.experimental import pallas as pl


def kernel(x, edge_index, edge_attr, weight_node, weight_edge, weight_triplet_att, weight_scale, bias):
    raise NotImplementedError("write your pallas kernel here")


# speedup vs baseline: 17.2199x; 17.2199x over previous
import jax
import jax.numpy as jnp
import numpy as np
from jax import lax
from jax.experimental import pallas as pl
from jax.experimental.pallas import tpu as pltpu
from jax.experimental.pallas import tpu_sc as plsc

N = 10000
N_PAD = 10240
E = 320000
D = 128
DE = 16
H = 3
HD = H * D

NC = 2
NS = 16
L = 16

_Z = np.int32(0)

CH = 128
NCH = E // CH
EPACK = E // L
DENW = N_PAD * H
RW = DENW // NS


_BN1 = 2048


def _a1_body(x_ref, wn_ref, watt_ref, xp0_ref, xp1_ref, xp2_ref, sds_ref):
    xb = x_ref[...]
    xp = jnp.dot(xb, wn_ref[...], preferred_element_type=jnp.float32)
    xp0_ref[...] = xp[:, 0 * D:1 * D]
    xp1_ref[...] = xp[:, 1 * D:2 * D]
    xp2_ref[...] = xp[:, 2 * D:3 * D]
    cols = []
    for h in range(H):
        wi = watt_ref[0, h, 0:D][None, :]
        cols.append(jnp.sum(xp[:, h * D:(h + 1) * D] * wi, axis=1,
                            keepdims=True))
    for h in range(H):
        wj = watt_ref[0, h, 2 * D:3 * D][None, :]
        cols.append(jnp.sum(xp[:, h * D:(h + 1) * D] * wj, axis=1,
                            keepdims=True))
    cols.append(jnp.zeros((xb.shape[0], 2), jnp.float32))
    sds_ref[...] = jnp.concatenate(cols, axis=1)


def _stage_a1(x_pad, wn, watt):
    grid = N_PAD // _BN1
    return pl.pallas_call(
        _a1_body,
        grid=(grid,),
        in_specs=[
            pl.BlockSpec((_BN1, D), lambda i: (i, _Z)),
            pl.BlockSpec((D, HD), lambda i: (_Z, _Z)),
            pl.BlockSpec((1, H, HD), lambda i: (_Z, _Z, _Z)),
        ],
        out_specs=[
            pl.BlockSpec((_BN1, D), lambda i: (i, _Z)),
            pl.BlockSpec((_BN1, D), lambda i: (i, _Z)),
            pl.BlockSpec((_BN1, D), lambda i: (i, _Z)),
            pl.BlockSpec((_BN1, 8), lambda i: (i, _Z)),
        ],
        out_shape=[
            jax.ShapeDtypeStruct((N_PAD, D), jnp.float32),
            jax.ShapeDtypeStruct((N_PAD, D), jnp.float32),
            jax.ShapeDtypeStruct((N_PAD, D), jnp.float32),
            jax.ShapeDtypeStruct((N_PAD, 8), jnp.float32),
        ],
    )(x_pad, wn, watt)


_BE = 2560


def _a2a_body(ea_ref, we_ref, watt_ref, se_ref):
    ep = jnp.dot(ea_ref[...], we_ref[...], preferred_element_type=jnp.float32)
    cols = []
    for h in range(H):
        wh = watt_ref[0, h, D:2 * D][None, :]
        cols.append(jnp.sum(ep[:, h * D:(h + 1) * D] * wh, axis=1,
                            keepdims=True))
    cols.append(jnp.zeros((ep.shape[0], 8 - H), jnp.float32))
    se_ref[...] = jnp.concatenate(cols, axis=1)


def _stage_a2a(ea, we, watt):
    grid = E // _BE
    return pl.pallas_call(
        _a2a_body,
        grid=(grid,),
        in_specs=[
            pl.BlockSpec((_BE, DE), lambda i: (i, _Z)),
            pl.BlockSpec((DE, HD), lambda i: (_Z, _Z)),
            pl.BlockSpec((1, H, HD), lambda i: (_Z, _Z, _Z)),
        ],
        out_specs=[pl.BlockSpec((_BE, 8), lambda i: (i, _Z))],
        out_shape=[jax.ShapeDtypeStruct((E, 8), jnp.float32)],
    )(ea, we, watt)[0]


def _sc1_body(src_hbm, dst_hbm, sds_hbm, se_hbm,
              ex_hbm, den0_hbm, den1_hbm,
              srcv, dstv, tab, sebuf, exbuf, denl, sem):
    c = lax.axis_index("c")
    s = lax.axis_index("s")
    wid = c * jnp.int32(NS) + s

    pltpu.sync_copy(sds_hbm, tab)

    def _z(i, _):
        denl[pl.ds(i * jnp.int32(L), L)] = jnp.zeros((L,), jnp.float32)
        return jnp.int32(0)
    lax.fori_loop(jnp.int32(0), jnp.int32(DENW // L), _z, jnp.int32(0))

    lanes = jnp.arange(L, dtype=jnp.int32)
    nch = (jnp.int32(NCH) - wid + jnp.int32(NC * NS - 1)) // jnp.int32(NC * NS)

    def _chunk(k, _):
        ch = wid + k * jnp.int32(NC * NS)
        e0 = ch * jnp.int32(CH)
        r0 = ch * jnp.int32(CH // L)
        pltpu.sync_copy(src_hbm.at[pl.ds(e0, CH)], srcv)
        pltpu.sync_copy(dst_hbm.at[pl.ds(e0, CH)], dstv)
        pltpu.sync_copy(se_hbm.at[pl.ds(r0, CH // L), :], sebuf)
        for g in range(CH // L):
            dstg = dstv[pl.ds(g * L, L)]
            srcg = srcv[pl.ds(g * L, L)]
            gv = jnp.full((L,), g, jnp.int32)
            for h in range(H):
                fi = dstg * 8 + h
                fj = srcg * 8 + (H + h)
                ce = lanes * 8 + h
                a = (plsc.load_gather(tab, [fi >> 7, fi & 127])
                     + plsc.load_gather(tab, [fj >> 7, fj & 127])
                     + plsc.load_gather(sebuf, [gv, ce]))
                a = jnp.where(a >= 0.0, a, 0.2 * a)
                ex = jnp.exp(a)
                plsc.store_scatter(exbuf, [gv, ce], ex)
                plsc.addupdate_scatter(denl, [dstg * H + h], ex)
        pltpu.sync_copy(exbuf, ex_hbm.at[pl.ds(r0, CH // L), :])
        return jnp.int32(0)

    lax.fori_loop(jnp.int32(0), nch, _chunk, jnp.int32(0))

    @pl.when(c == 0)
    def _():
        pltpu.sync_copy(denl, den0_hbm.at[pl.ds(s * jnp.int32(DENW), DENW)])

    @pl.when(c == 1)
    def _():
        pltpu.sync_copy(denl, den1_hbm.at[pl.ds(s * jnp.int32(DENW), DENW)])


def _stage_sc1(src, dst, sds, se):
    mesh = plsc.VectorSubcoreMesh(core_axis_name="c", subcore_axis_name="s")
    f = pl.kernel(
        _sc1_body,
        mesh=mesh,
        out_type=[
            jax.ShapeDtypeStruct((EPACK, 8 * L), jnp.float32),
            jax.ShapeDtypeStruct((NS * DENW,), jnp.float32),
            jax.ShapeDtypeStruct((NS * DENW,), jnp.float32),
        ],
        compiler_params=pltpu.CompilerParams(needs_layout_passes=False),
        scratch_types=[
            pltpu.VMEM((CH,), jnp.int32),
            pltpu.VMEM((CH,), jnp.int32),
            pltpu.VMEM((N_PAD // L, 8 * L), jnp.float32),
            pltpu.VMEM((CH // L, 8 * L), jnp.float32),
            pltpu.VMEM((CH // L, 8 * L), jnp.float32),
            pltpu.VMEM((DENW,), jnp.float32),
            pltpu.SemaphoreType.DMA,
        ],
    )
    return f(src, dst, sds, se)


def _a2b_body(ea_ref, we_ref, ex_ref, epx0_ref, epx1_ref, epx2_ref):
    ep = jnp.dot(ea_ref[...], we_ref[...], preferred_element_type=jnp.float32)
    exb = ex_ref[...]
    outs = (epx0_ref, epx1_ref, epx2_ref)
    for h in range(H):
        outs[h][...] = ep[:, h * D:(h + 1) * D] * exb[:, h:h + 1]


def _stage_a2b(ea, we, ex):
    grid = E // _BE
    return pl.pallas_call(
        _a2b_body,
        grid=(grid,),
        in_specs=[
            pl.BlockSpec((_BE, DE), lambda i: (i, _Z)),
            pl.BlockSpec((DE, HD), lambda i: (_Z, _Z)),
            pl.BlockSpec((_BE, 8), lambda i: (i, _Z)),
        ],
        out_specs=[
            pl.BlockSpec((_BE, D), lambda i: (i, _Z)),
            pl.BlockSpec((_BE, D), lambda i: (i, _Z)),
            pl.BlockSpec((_BE, D), lambda i: (i, _Z)),
        ],
        out_shape=[
            jax.ShapeDtypeStruct((E, D), jnp.float32),
            jax.ShapeDtypeStruct((E, D), jnp.float32),
            jax.ShapeDtypeStruct((E, D), jnp.float32),
        ],
    )(ea, we, ex)


_NROWS_T = N_PAD // NS
_ZCH = _NROWS_T // CH


def _sc2_body(src_hbm, dst_hbm, xp0_hbm, xp1_hbm, xp2_hbm,
              ep0_hbm, ep1_hbm, ep2_hbm, den0_hbm, den1_hbm,
              ag0_hbm, ag1a_hbm, ag1b_hbm, ag2_hbm, inv_hbm,
              srcv, dstv, xprows, eprows, acc, tmp, invbuf,
              agsp, sem):
    c = lax.axis_index("c")
    s = lax.axis_index("s")
    lanes = jnp.arange(L, dtype=jnp.int32)

    def _zz(i, _):
        acc[pl.ds(i * jnp.int32(L), L)] = jnp.zeros((L,), jnp.float32)
        return jnp.int32(0)
    lax.fori_loop(jnp.int32(0), jnp.int32(RW // L), _zz, jnp.int32(0))
    for part in (den0_hbm, den1_hbm):
        for t in range(NS):
            pltpu.sync_copy(
                part.at[pl.ds(jnp.int32(t * DENW) + s * jnp.int32(RW), RW)],
                tmp)

            def _red(i, _):
                sl = pl.ds(i * jnp.int32(L), L)
                acc[sl] = acc[sl] + tmp[sl]
                return jnp.int32(0)
            lax.fori_loop(jnp.int32(0), jnp.int32(RW // L), _red, jnp.int32(0))

    def _inv(b, _):
        for h in range(H):
            idx = (b * jnp.int32(L) + lanes) * jnp.int32(H) + h
            v = plsc.load_gather(acc, [idx])
            iv = 1.0 / jnp.maximum(v, 1e-16)
            plsc.store_scatter(invbuf, [jnp.full((L,), 0, jnp.int32) + b,
                                        lanes * 8 + h], iv)
        return jnp.int32(0)
    lax.fori_loop(jnp.int32(0), jnp.int32(RW // (L * H)), _inv, jnp.int32(0))

    @pl.when(c == 0)
    def _():
        pltpu.sync_copy(
            invbuf, inv_hbm.at[pl.ds(s * jnp.int32(RW // (L * H)),
                                     RW // (L * H)), :])

    def _zero_aggr():
        def _zr(r, _):
            for j in range(D // L):
                xprows[r, pl.ds(j * L, L)] = jnp.zeros((L,), jnp.float32)
            return jnp.int32(0)
        lax.fori_loop(jnp.int32(0), jnp.int32(CH), _zr, jnp.int32(0))
        for k in range(_ZCH):
            pltpu.sync_copy(
                xprows,
                agsp.at[pl.ds(s * jnp.int32(_NROWS_T) + jnp.int32(k * CH),
                              CH), :])

    def _copy_out(dst_ref):
        for k in range(_ZCH):
            r0 = s * jnp.int32(_NROWS_T) + jnp.int32(k * CH)
            pltpu.sync_copy(agsp.at[pl.ds(r0, CH), :],
                            dst_ref.at[pl.ds(r0, CH), :])

    def _run(xp_hbm, ep_hbm, ch0, nch):
        def _chunk(k, _):
            ch = ch0 + k * jnp.int32(NS)
            e0 = ch * jnp.int32(CH)
            pltpu.sync_copy(src_hbm.at[pl.ds(e0, CH)], srcv)
            pltpu.sync_copy(dst_hbm.at[pl.ds(e0, CH)], dstv)
            pltpu.async_copy(xp_hbm.at[srcv], xprows, sem).wait()
            pltpu.sync_copy(ep_hbm.at[pl.ds(e0, CH), :], eprows)

            def _row(r, _2):
                for j in range(D // L):
                    sl = pl.ds(j * L, L)
                    eprows[r, sl] = eprows[r, sl] * xprows[r, sl]
                return jnp.int32(0)
            lax.fori_loop(jnp.int32(0), jnp.int32(CH), _row, jnp.int32(0))

            pltpu.sync_copy(eprows, agsp.at[dstv], add=True)
            return jnp.int32(0)
        lax.fori_loop(jnp.int32(0), nch, _chunk, jnp.int32(0))

    _zero_aggr()
    plsc.subcore_barrier()

    ncha = (jnp.int32(NCH) - s + jnp.int32(NS - 1)) // jnp.int32(NS)

    @pl.when(c == 0)
    def _():
        _run(xp0_hbm, ep0_hbm, s, ncha)

    @pl.when(c == 1)
    def _():
        _run(xp2_hbm, ep2_hbm, s, ncha)

    plsc.subcore_barrier()

    @pl.when(c == 0)
    def _():
        _copy_out(ag0_hbm)

    @pl.when(c == 1)
    def _():
        _copy_out(ag2_hbm)

    plsc.subcore_barrier()
    _zero_aggr()
    plsc.subcore_barrier()

    nchb = (jnp.int32(NCH // 2) - s + jnp.int32(NS - 1)) // jnp.int32(NS)

    @pl.when(c == 0)
    def _():
        _run(xp1_hbm, ep1_hbm, s, nchb)

    @pl.when(c == 1)
    def _():
        _run(xp1_hbm, ep1_hbm, jnp.int32(NCH // 2) + s, nchb)

    plsc.subcore_barrier()

    @pl.when(c == 0)
    def _():
        _copy_out(ag1a_hbm)

    @pl.when(c == 1)
    def _():
        _copy_out(ag1b_hbm)


def _stage_sc2(src, dst, xp0, xp1, xp2, ep0, ep1, ep2, den0, den1):
    mesh = plsc.VectorSubcoreMesh(core_axis_name="c", subcore_axis_name="s")
    f = pl.kernel(
        _sc2_body,
        mesh=mesh,
        out_type=[
            jax.ShapeDtypeStruct((N_PAD, D), jnp.float32),
            jax.ShapeDtypeStruct((N_PAD, D), jnp.float32),
            jax.ShapeDtypeStruct((N_PAD, D), jnp.float32),
            jax.ShapeDtypeStruct((N_PAD, D), jnp.float32),
            jax.ShapeDtypeStruct((N_PAD // L, 8 * L), jnp.float32),
        ],
        compiler_params=pltpu.CompilerParams(needs_layout_passes=False),
        scratch_types=[
            pltpu.VMEM((CH,), jnp.int32),
            pltpu.VMEM((CH,), jnp.int32),
            pltpu.VMEM((CH, D), jnp.float32),
            pltpu.VMEM((CH, D), jnp.float32),
            pltpu.VMEM((RW,), jnp.float32),
            pltpu.VMEM((RW,), jnp.float32),
            pltpu.VMEM((RW // (L * H), 8 * L), jnp.float32),
            pltpu.VMEM_SHARED((N_PAD, D), jnp.float32),
            pltpu.SemaphoreType.DMA,
        ],
    )
    return f(src, dst, xp0, xp1, xp2, ep0, ep1, ep2, den0, den1)


_BNF = 1024
_BDF = _BNF * H // (8 * L)


def _final_body(a0_ref, a1a_ref, a1b_ref, a2_ref, inv_ref,
                ws_ref, b_ref, out_ref):
    inv = inv_ref[...]
    heads = (a0_ref[...], a1a_ref[...] + a1b_ref[...], a2_ref[...])
    acc = b_ref[...]
    for h in range(H):
        scaled = heads[h] * inv[:, h:h + 1]
        acc = acc + jnp.dot(scaled, ws_ref[h * D:(h + 1) * D, :],
                            preferred_element_type=jnp.float32)
    out_ref[...] = acc


def _stage_final(a0, a1a, a1b, a2, inv, ws, bias2d):
    grid = N_PAD // _BNF
    return pl.pallas_call(
        _final_body,
        grid=(grid,),
        in_specs=[
            pl.BlockSpec((_BNF, D), lambda i: (i, _Z)),
            pl.BlockSpec((_BNF, D), lambda i: (i, _Z)),
            pl.BlockSpec((_BNF, D), lambda i: (i, _Z)),
            pl.BlockSpec((_BNF, D), lambda i: (i, _Z)),
            pl.BlockSpec((_BNF, 8), lambda i: (i, _Z)),
            pl.BlockSpec((HD, D), lambda i: (_Z, _Z)),
            pl.BlockSpec((1, D), lambda i: (_Z, _Z)),
        ],
        out_specs=[pl.BlockSpec((_BNF, D), lambda i: (i, _Z))],
        out_shape=[jax.ShapeDtypeStruct((N_PAD, D), jnp.float32)],
    )(a0, a1a, a1b, a2, inv, ws, bias2d)[0]


def kernel(x, edge_index, edge_attr, weight_node, weight_edge,
           weight_triplet_att, weight_scale, bias):
    src = edge_index[0].astype(jnp.int32)
    dst = edge_index[1].astype(jnp.int32)
    x_pad = jnp.concatenate(
        [x.astype(jnp.float32),
         jnp.zeros((N_PAD - N, D), jnp.float32)], axis=0)
    ea = edge_attr.astype(jnp.float32)

    xp0, xp1, xp2, sds = _stage_a1(x_pad, weight_node, weight_triplet_att)
    se = _stage_a2a(ea, weight_edge, weight_triplet_att)
    ex, den0, den1 = _stage_sc1(src, dst,
                                sds.reshape(N_PAD // L, 8 * L),
                                se.reshape(EPACK, 8 * L))
    ep0, ep1, ep2 = _stage_a2b(ea, weight_edge, ex.reshape(E, 8))
    a0, a1a, a1b, a2, invden = _stage_sc2(src, dst, xp0, xp1, xp2,
                                          ep0, ep1, ep2, den0, den1)
    out = _stage_final(a0, a1a, a1b, a2, invden.reshape(N_PAD, 8),
                       weight_scale,
                       bias.reshape(1, D).astype(jnp.float32))
    return out[:N]


# speedup vs baseline: 21.4543x; 1.2459x over previous
import jax
import jax.numpy as jnp
import numpy as np
from jax import lax
from jax.experimental import pallas as pl
from jax.experimental.pallas import tpu as pltpu
from jax.experimental.pallas import tpu_sc as plsc

N = 10000
N_PAD = 10240
E = 320000
D = 128
DE = 16
H = 3
HD = H * D

NC = 2
NS = 16
L = 16

_Z = np.int32(0)

CH = 128
NCH = E // CH
EPACK = E // L
DENW = N_PAD * H
RW = DENW // NS


_BN1 = 2048


def _a1_body(x_ref, wn_ref, watt_ref, xp0_ref, xp1_ref, xp2_ref, sds_ref):
    xb = x_ref[...]
    xp = jnp.dot(xb, wn_ref[...], preferred_element_type=jnp.float32)
    xp0_ref[...] = xp[:, 0 * D:1 * D]
    xp1_ref[...] = xp[:, 1 * D:2 * D]
    xp2_ref[...] = xp[:, 2 * D:3 * D]
    cols = []
    for h in range(H):
        wi = watt_ref[0, h, 0:D][None, :]
        cols.append(jnp.sum(xp[:, h * D:(h + 1) * D] * wi, axis=1,
                            keepdims=True))
    for h in range(H):
        wj = watt_ref[0, h, 2 * D:3 * D][None, :]
        cols.append(jnp.sum(xp[:, h * D:(h + 1) * D] * wj, axis=1,
                            keepdims=True))
    cols.append(jnp.zeros((xb.shape[0], 2), jnp.float32))
    sds_ref[...] = jnp.concatenate(cols, axis=1)


def _stage_a1(x_pad, wn, watt):
    grid = N_PAD // _BN1
    return pl.pallas_call(
        _a1_body,
        grid=(grid,),
        in_specs=[
            pl.BlockSpec((_BN1, D), lambda i: (i, _Z)),
            pl.BlockSpec((D, HD), lambda i: (_Z, _Z)),
            pl.BlockSpec((1, H, HD), lambda i: (_Z, _Z, _Z)),
        ],
        out_specs=[
            pl.BlockSpec((_BN1, D), lambda i: (i, _Z)),
            pl.BlockSpec((_BN1, D), lambda i: (i, _Z)),
            pl.BlockSpec((_BN1, D), lambda i: (i, _Z)),
            pl.BlockSpec((_BN1, 8), lambda i: (i, _Z)),
        ],
        out_shape=[
            jax.ShapeDtypeStruct((N_PAD, D), jnp.float32),
            jax.ShapeDtypeStruct((N_PAD, D), jnp.float32),
            jax.ShapeDtypeStruct((N_PAD, D), jnp.float32),
            jax.ShapeDtypeStruct((N_PAD, 8), jnp.float32),
        ],
    )(x_pad, wn, watt)


_BE = 2560


def _a2a_body(ea_ref, we_ref, watt_ref, se_ref):
    ep = jnp.dot(ea_ref[...], we_ref[...], preferred_element_type=jnp.float32)
    cols = []
    for h in range(H):
        wh = watt_ref[0, h, D:2 * D][None, :]
        cols.append(jnp.sum(ep[:, h * D:(h + 1) * D] * wh, axis=1,
                            keepdims=True))
    cols.append(jnp.zeros((ep.shape[0], 8 - H), jnp.float32))
    se_ref[...] = jnp.concatenate(cols, axis=1)


def _stage_a2a(ea, we, watt):
    grid = E // _BE
    return pl.pallas_call(
        _a2a_body,
        grid=(grid,),
        in_specs=[
            pl.BlockSpec((_BE, DE), lambda i: (i, _Z)),
            pl.BlockSpec((DE, HD), lambda i: (_Z, _Z)),
            pl.BlockSpec((1, H, HD), lambda i: (_Z, _Z, _Z)),
        ],
        out_specs=[pl.BlockSpec((_BE, 8), lambda i: (i, _Z))],
        out_shape=[jax.ShapeDtypeStruct((E, 8), jnp.float32)],
    )(ea, we, watt)[0]


def _sc1_body(src_hbm, dst_hbm, sds_hbm, se_hbm,
              ex_hbm, den0_hbm, den1_hbm,
              srcv, dstv, tab, sebuf, exbuf, denl, sem):
    c = lax.axis_index("c")
    s = lax.axis_index("s")
    wid = c * jnp.int32(NS) + s

    pltpu.sync_copy(sds_hbm, tab)

    def _z(i, _):
        denl[pl.ds(i * jnp.int32(L), L)] = jnp.zeros((L,), jnp.float32)
        return jnp.int32(0)
    lax.fori_loop(jnp.int32(0), jnp.int32(DENW // L), _z, jnp.int32(0))

    lanes = jnp.arange(L, dtype=jnp.int32)
    nch = (jnp.int32(NCH) - wid + jnp.int32(NC * NS - 1)) // jnp.int32(NC * NS)

    def _chunk(k, _):
        ch = wid + k * jnp.int32(NC * NS)
        e0 = ch * jnp.int32(CH)
        r0 = ch * jnp.int32(CH // L)
        pltpu.sync_copy(src_hbm.at[pl.ds(e0, CH)], srcv)
        pltpu.sync_copy(dst_hbm.at[pl.ds(e0, CH)], dstv)
        pltpu.sync_copy(se_hbm.at[pl.ds(r0, CH // L), :], sebuf)
        for g in range(CH // L):
            dstg = dstv[pl.ds(g * L, L)]
            srcg = srcv[pl.ds(g * L, L)]
            gv = jnp.full((L,), g, jnp.int32)
            for h in range(H):
                fi = dstg * 8 + h
                fj = srcg * 8 + (H + h)
                ce = lanes * 8 + h
                a = (plsc.load_gather(tab, [fi >> 7, fi & 127])
                     + plsc.load_gather(tab, [fj >> 7, fj & 127])
                     + plsc.load_gather(sebuf, [gv, ce]))
                a = jnp.where(a >= 0.0, a, 0.2 * a)
                ex = jnp.exp(a)
                plsc.store_scatter(exbuf, [gv, ce], ex)
                plsc.addupdate_scatter(denl, [dstg * H + h], ex)
        pltpu.sync_copy(exbuf, ex_hbm.at[pl.ds(r0, CH // L), :])
        return jnp.int32(0)

    lax.fori_loop(jnp.int32(0), nch, _chunk, jnp.int32(0))

    @pl.when(c == 0)
    def _():
        pltpu.sync_copy(denl, den0_hbm.at[pl.ds(s * jnp.int32(DENW), DENW)])

    @pl.when(c == 1)
    def _():
        pltpu.sync_copy(denl, den1_hbm.at[pl.ds(s * jnp.int32(DENW), DENW)])


def _stage_sc1(src, dst, sds, se):
    mesh = plsc.VectorSubcoreMesh(core_axis_name="c", subcore_axis_name="s")
    f = pl.kernel(
        _sc1_body,
        mesh=mesh,
        out_type=[
            jax.ShapeDtypeStruct((EPACK, 8 * L), jnp.float32),
            jax.ShapeDtypeStruct((NS * DENW,), jnp.float32),
            jax.ShapeDtypeStruct((NS * DENW,), jnp.float32),
        ],
        compiler_params=pltpu.CompilerParams(needs_layout_passes=False),
        scratch_types=[
            pltpu.VMEM((CH,), jnp.int32),
            pltpu.VMEM((CH,), jnp.int32),
            pltpu.VMEM((N_PAD // L, 8 * L), jnp.float32),
            pltpu.VMEM((CH // L, 8 * L), jnp.float32),
            pltpu.VMEM((CH // L, 8 * L), jnp.float32),
            pltpu.VMEM((DENW,), jnp.float32),
            pltpu.SemaphoreType.DMA,
        ],
    )
    return f(src, dst, sds, se)


def _a2b_body(ea_ref, we_ref, ex_ref, epx0_ref, epx1_ref, epx2_ref):
    ep = jnp.dot(ea_ref[...], we_ref[...], preferred_element_type=jnp.float32)
    exb = ex_ref[...]
    outs = (epx0_ref, epx1_ref, epx2_ref)
    for h in range(H):
        outs[h][...] = ep[:, h * D:(h + 1) * D] * exb[:, h:h + 1]


def _stage_a2b(ea, we, ex):
    grid = E // _BE
    return pl.pallas_call(
        _a2b_body,
        grid=(grid,),
        in_specs=[
            pl.BlockSpec((_BE, DE), lambda i: (i, _Z)),
            pl.BlockSpec((DE, HD), lambda i: (_Z, _Z)),
            pl.BlockSpec((_BE, 8), lambda i: (i, _Z)),
        ],
        out_specs=[
            pl.BlockSpec((_BE, D), lambda i: (i, _Z)),
            pl.BlockSpec((_BE, D), lambda i: (i, _Z)),
            pl.BlockSpec((_BE, D), lambda i: (i, _Z)),
        ],
        out_shape=[
            jax.ShapeDtypeStruct((E, D), jnp.float32),
            jax.ShapeDtypeStruct((E, D), jnp.float32),
            jax.ShapeDtypeStruct((E, D), jnp.float32),
        ],
    )(ea, we, ex)


_NROWS_T = N_PAD // NS
CH2 = 64
NCH2 = E // CH2
_ZCH = _NROWS_T // CH


def _sc2_body(src_hbm, dst_hbm, xp0_hbm, xp1_hbm, xp2_hbm,
              ep0_hbm, ep1_hbm, ep2_hbm, den0_hbm, den1_hbm,
              ag0_hbm, ag1a_hbm, ag1b_hbm, ag2_hbm, inv_hbm,
              srcv0, dstv0, xpr0, epr0, srcv1, dstv1, xpr1, epr1,
              acc, tmp, invbuf,
              agsp, gsem0, ssem0, gsem1, ssem1):
    c = lax.axis_index("c")
    s = lax.axis_index("s")
    lanes = jnp.arange(L, dtype=jnp.int32)

    def _zz(i, _):
        acc[pl.ds(i * jnp.int32(L), L)] = jnp.zeros((L,), jnp.float32)
        return jnp.int32(0)
    lax.fori_loop(jnp.int32(0), jnp.int32(RW // L), _zz, jnp.int32(0))
    for part in (den0_hbm, den1_hbm):
        for t in range(NS):
            pltpu.sync_copy(
                part.at[pl.ds(jnp.int32(t * DENW) + s * jnp.int32(RW), RW)],
                tmp)

            def _red(i, _):
                sl = pl.ds(i * jnp.int32(L), L)
                acc[sl] = acc[sl] + tmp[sl]
                return jnp.int32(0)
            lax.fori_loop(jnp.int32(0), jnp.int32(RW // L), _red, jnp.int32(0))

    def _inv(b, _):
        for h in range(H):
            idx = (b * jnp.int32(L) + lanes) * jnp.int32(H) + h
            v = plsc.load_gather(acc, [idx])
            iv = 1.0 / jnp.maximum(v, 1e-16)
            plsc.store_scatter(invbuf, [jnp.full((L,), 0, jnp.int32) + b,
                                        lanes * 8 + h], iv)
        return jnp.int32(0)
    lax.fori_loop(jnp.int32(0), jnp.int32(RW // (L * H)), _inv, jnp.int32(0))

    @pl.when(c == 0)
    def _():
        pltpu.sync_copy(
            invbuf, inv_hbm.at[pl.ds(s * jnp.int32(RW // (L * H)),
                                     RW // (L * H)), :])

    def _zero_aggr():
        def _zr(r, _):
            for j in range(D // L):
                xpr0[r, pl.ds(j * L, L)] = jnp.zeros((L,), jnp.float32)
            return jnp.int32(0)
        lax.fori_loop(jnp.int32(0), jnp.int32(CH2), _zr, jnp.int32(0))
        for k in range(_NROWS_T // CH2):
            pltpu.sync_copy(
                xpr0,
                agsp.at[pl.ds(s * jnp.int32(_NROWS_T) + jnp.int32(k * CH2),
                              CH2), :])

    def _copy_out(dst_ref):
        for k in range(_ZCH):
            r0 = s * jnp.int32(_NROWS_T) + jnp.int32(k * CH)
            pltpu.sync_copy(agsp.at[pl.ds(r0, CH), :],
                            dst_ref.at[pl.ds(r0, CH), :])

    bufs = ((srcv0, dstv0, xpr0, epr0, gsem0, ssem0),
            (srcv1, dstv1, xpr1, epr1, gsem1, ssem1))

    def _run(xp_hbm, ep_hbm, ch0, nch):
        def _e0(i):
            return (ch0 + i * jnp.int32(NS)) * jnp.int32(CH2)

        def _fetch(i, b):
            sv, dv, xr, er, gs, ss = bufs[b]
            pltpu.sync_copy(src_hbm.at[pl.ds(_e0(i), CH2)], sv)
            pltpu.sync_copy(dst_hbm.at[pl.ds(_e0(i), CH2)], dv)
            pltpu.async_copy(xp_hbm.at[sv], xr, gs)
            pltpu.async_copy(ep_hbm.at[pl.ds(_e0(i), CH2), :], er, gs)

        def _drain_scatter(b):
            sv, dv, xr, er, gs, ss = bufs[b]
            pltpu.make_async_copy(er, agsp.at[dv], ss).wait()

        def _process(i, b):
            sv, dv, xr, er, gs, ss = bufs[b]
            pltpu.make_async_copy(xp_hbm.at[sv], xr, gs).wait()
            pltpu.make_async_copy(ep_hbm.at[pl.ds(_e0(i), CH2), :],
                                  er, gs).wait()

            def _row(r2, _2):
                for rr in range(2):
                    r = r2 * jnp.int32(2) + rr
                    for j in range(D // L):
                        sl = pl.ds(j * L, L)
                        er[r, sl] = er[r, sl] * xr[r, sl]
                return jnp.int32(0)
            lax.fori_loop(jnp.int32(0), jnp.int32(CH2 // 2), _row,
                          jnp.int32(0))
            pltpu.async_copy(er, agsp.at[dv], ss, add=True)

        _fetch(jnp.int32(0), 0)
        npair = (nch + jnp.int32(1)) // jnp.int32(2)

        def _pair(k2, _):
            i0 = k2 * jnp.int32(2)
            for b in (0, 1):
                i = i0 + b
                nb = 1 - b

                @pl.when(i < nch)
                def _():
                    @pl.when(i + jnp.int32(1) < nch)
                    def _():
                        @pl.when(i >= jnp.int32(1))
                        def _():
                            _drain_scatter(nb)
                        _fetch(i + jnp.int32(1), nb)
                    _process(i, b)
            return jnp.int32(0)
        lax.fori_loop(jnp.int32(0), npair, _pair, jnp.int32(0))

        _drain_scatter(0)
        _drain_scatter(1)

    _zero_aggr()
    plsc.subcore_barrier()

    ncha = (jnp.int32(NCH2) - s + jnp.int32(NS - 1)) // jnp.int32(NS)

    @pl.when(c == 0)
    def _():
        _run(xp0_hbm, ep0_hbm, s, ncha)

    @pl.when(c == 1)
    def _():
        _run(xp2_hbm, ep2_hbm, s, ncha)

    plsc.subcore_barrier()

    @pl.when(c == 0)
    def _():
        _copy_out(ag0_hbm)

    @pl.when(c == 1)
    def _():
        _copy_out(ag2_hbm)

    plsc.subcore_barrier()
    _zero_aggr()
    plsc.subcore_barrier()

    nchb = (jnp.int32(NCH2 // 2) - s + jnp.int32(NS - 1)) // jnp.int32(NS)

    @pl.when(c == 0)
    def _():
        _run(xp1_hbm, ep1_hbm, s, nchb)

    @pl.when(c == 1)
    def _():
        _run(xp1_hbm, ep1_hbm, jnp.int32(NCH2 // 2) + s, nchb)

    plsc.subcore_barrier()

    @pl.when(c == 0)
    def _():
        _copy_out(ag1a_hbm)

    @pl.when(c == 1)
    def _():
        _copy_out(ag1b_hbm)


def _stage_sc2(src, dst, xp0, xp1, xp2, ep0, ep1, ep2, den0, den1):
    mesh = plsc.VectorSubcoreMesh(core_axis_name="c", subcore_axis_name="s")
    f = pl.kernel(
        _sc2_body,
        mesh=mesh,
        out_type=[
            jax.ShapeDtypeStruct((N_PAD, D), jnp.float32),
            jax.ShapeDtypeStruct((N_PAD, D), jnp.float32),
            jax.ShapeDtypeStruct((N_PAD, D), jnp.float32),
            jax.ShapeDtypeStruct((N_PAD, D), jnp.float32),
            jax.ShapeDtypeStruct((N_PAD // L, 8 * L), jnp.float32),
        ],
        compiler_params=pltpu.CompilerParams(needs_layout_passes=False),
        scratch_types=[
            pltpu.VMEM((CH2,), jnp.int32),
            pltpu.VMEM((CH2,), jnp.int32),
            pltpu.VMEM((CH2, D), jnp.float32),
            pltpu.VMEM((CH2, D), jnp.float32),
            pltpu.VMEM((CH2,), jnp.int32),
            pltpu.VMEM((CH2,), jnp.int32),
            pltpu.VMEM((CH2, D), jnp.float32),
            pltpu.VMEM((CH2, D), jnp.float32),
            pltpu.VMEM((RW,), jnp.float32),
            pltpu.VMEM((RW,), jnp.float32),
            pltpu.VMEM((RW // (L * H), 8 * L), jnp.float32),
            pltpu.VMEM_SHARED((N_PAD, D), jnp.float32),
            pltpu.SemaphoreType.DMA,
            pltpu.SemaphoreType.DMA,
            pltpu.SemaphoreType.DMA,
            pltpu.SemaphoreType.DMA,
        ],
    )
    return f(src, dst, xp0, xp1, xp2, ep0, ep1, ep2, den0, den1)


_BNF = 1024
_BDF = _BNF * H // (8 * L)


def _final_body(a0_ref, a1a_ref, a1b_ref, a2_ref, inv_ref,
                ws_ref, b_ref, out_ref):
    inv = inv_ref[...]
    heads = (a0_ref[...], a1a_ref[...] + a1b_ref[...], a2_ref[...])
    acc = b_ref[...]
    for h in range(H):
        scaled = heads[h] * inv[:, h:h + 1]
        acc = acc + jnp.dot(scaled, ws_ref[h * D:(h + 1) * D, :],
                            preferred_element_type=jnp.float32)
    out_ref[...] = acc


def _stage_final(a0, a1a, a1b, a2, inv, ws, bias2d):
    grid = N_PAD // _BNF
    return pl.pallas_call(
        _final_body,
        grid=(grid,),
        in_specs=[
            pl.BlockSpec((_BNF, D), lambda i: (i, _Z)),
            pl.BlockSpec((_BNF, D), lambda i: (i, _Z)),
            pl.BlockSpec((_BNF, D), lambda i: (i, _Z)),
            pl.BlockSpec((_BNF, D), lambda i: (i, _Z)),
            pl.BlockSpec((_BNF, 8), lambda i: (i, _Z)),
            pl.BlockSpec((HD, D), lambda i: (_Z, _Z)),
            pl.BlockSpec((1, D), lambda i: (_Z, _Z)),
        ],
        out_specs=[pl.BlockSpec((_BNF, D), lambda i: (i, _Z))],
        out_shape=[jax.ShapeDtypeStruct((N_PAD, D), jnp.float32)],
    )(a0, a1a, a1b, a2, inv, ws, bias2d)[0]


def kernel(x, edge_index, edge_attr, weight_node, weight_edge,
           weight_triplet_att, weight_scale, bias):
    src = edge_index[0].astype(jnp.int32)
    dst = edge_index[1].astype(jnp.int32)
    x_pad = jnp.concatenate(
        [x.astype(jnp.float32),
         jnp.zeros((N_PAD - N, D), jnp.float32)], axis=0)
    ea = edge_attr.astype(jnp.float32)

    xp0, xp1, xp2, sds = _stage_a1(x_pad, weight_node, weight_triplet_att)
    se = _stage_a2a(ea, weight_edge, weight_triplet_att)
    ex, den0, den1 = _stage_sc1(src, dst,
                                sds.reshape(N_PAD // L, 8 * L),
                                se.reshape(EPACK, 8 * L))
    ep0, ep1, ep2 = _stage_a2b(ea, weight_edge, ex.reshape(E, 8))
    a0, a1a, a1b, a2, invden = _stage_sc2(src, dst, xp0, xp1, xp2,
                                          ep0, ep1, ep2, den0, den1)
    out = _stage_final(a0, a1a, a1b, a2, invden.reshape(N_PAD, 8),
                       weight_scale,
                       bias.reshape(1, D).astype(jnp.float32))
    return out[:N]


# speedup vs baseline: 22.6411x; 1.0553x over previous
import jax
import jax.numpy as jnp
import numpy as np
from jax import lax
from jax.experimental import pallas as pl
from jax.experimental.pallas import tpu as pltpu
from jax.experimental.pallas import tpu_sc as plsc

N = 10000
N_PAD = 10240
E = 320000
D = 128
DE = 16
H = 3
HD = H * D

NC = 2
NS = 16
L = 16

_Z = np.int32(0)

CH = 128
NCH = E // CH
EPACK = E // L
DENW = N_PAD * H
RW = DENW // NS


_BN1 = 2048


def _a1_body(x_ref, wn_ref, watt_ref, xp0_ref, xp1_ref, xp2_ref, sds_ref):
    xb = x_ref[...]
    xp = jnp.dot(xb, wn_ref[...], preferred_element_type=jnp.float32)
    xp0_ref[...] = xp[:, 0 * D:1 * D]
    xp1_ref[...] = xp[:, 1 * D:2 * D]
    xp2_ref[...] = xp[:, 2 * D:3 * D]
    cols = []
    for h in range(H):
        wi = watt_ref[0, h, 0:D][None, :]
        cols.append(jnp.sum(xp[:, h * D:(h + 1) * D] * wi, axis=1,
                            keepdims=True))
    for h in range(H):
        wj = watt_ref[0, h, 2 * D:3 * D][None, :]
        cols.append(jnp.sum(xp[:, h * D:(h + 1) * D] * wj, axis=1,
                            keepdims=True))
    cols.append(jnp.zeros((xb.shape[0], 2), jnp.float32))
    sds_ref[...] = jnp.concatenate(cols, axis=1)


def _stage_a1(x_pad, wn, watt):
    grid = N_PAD // _BN1
    return pl.pallas_call(
        _a1_body,
        grid=(grid,),
        in_specs=[
            pl.BlockSpec((_BN1, D), lambda i: (i, _Z)),
            pl.BlockSpec((D, HD), lambda i: (_Z, _Z)),
            pl.BlockSpec((1, H, HD), lambda i: (_Z, _Z, _Z)),
        ],
        out_specs=[
            pl.BlockSpec((_BN1, D), lambda i: (i, _Z)),
            pl.BlockSpec((_BN1, D), lambda i: (i, _Z)),
            pl.BlockSpec((_BN1, D), lambda i: (i, _Z)),
            pl.BlockSpec((_BN1, 8), lambda i: (i, _Z)),
        ],
        out_shape=[
            jax.ShapeDtypeStruct((N_PAD, D), jnp.float32),
            jax.ShapeDtypeStruct((N_PAD, D), jnp.float32),
            jax.ShapeDtypeStruct((N_PAD, D), jnp.float32),
            jax.ShapeDtypeStruct((N_PAD, 8), jnp.float32),
        ],
    )(x_pad, wn, watt)


_BE = 2560


def _a2a_body(ea_ref, we_ref, watt_ref, se_ref):
    ep = jnp.dot(ea_ref[...], we_ref[...], preferred_element_type=jnp.float32)
    cols = []
    for h in range(H):
        wh = watt_ref[0, h, D:2 * D][None, :]
        cols.append(jnp.sum(ep[:, h * D:(h + 1) * D] * wh, axis=1,
                            keepdims=True))
    cols.append(jnp.zeros((ep.shape[0], 8 - H), jnp.float32))
    se_ref[...] = jnp.concatenate(cols, axis=1)


def _stage_a2a(ea, we, watt):
    grid = E // _BE
    return pl.pallas_call(
        _a2a_body,
        grid=(grid,),
        in_specs=[
            pl.BlockSpec((_BE, DE), lambda i: (i, _Z)),
            pl.BlockSpec((DE, HD), lambda i: (_Z, _Z)),
            pl.BlockSpec((1, H, HD), lambda i: (_Z, _Z, _Z)),
        ],
        out_specs=[pl.BlockSpec((_BE, 8), lambda i: (i, _Z))],
        out_shape=[jax.ShapeDtypeStruct((E, 8), jnp.float32)],
    )(ea, we, watt)[0]


def _sc1_body(src_hbm, dst_hbm, sds_hbm, se_hbm,
              ex_hbm, den0_hbm, den1_hbm,
              srcv, dstv, tab, sebuf, exbuf, denl, sem):
    c = lax.axis_index("c")
    s = lax.axis_index("s")
    wid = c * jnp.int32(NS) + s

    pltpu.sync_copy(sds_hbm, tab)

    def _z(i, _):
        denl[pl.ds(i * jnp.int32(L), L)] = jnp.zeros((L,), jnp.float32)
        return jnp.int32(0)
    lax.fori_loop(jnp.int32(0), jnp.int32(DENW // L), _z, jnp.int32(0))

    lanes = jnp.arange(L, dtype=jnp.int32)
    nch = (jnp.int32(NCH) - wid + jnp.int32(NC * NS - 1)) // jnp.int32(NC * NS)

    def _chunk(k, _):
        ch = wid + k * jnp.int32(NC * NS)
        e0 = ch * jnp.int32(CH)
        r0 = ch * jnp.int32(CH // L)
        pltpu.sync_copy(src_hbm.at[pl.ds(e0, CH)], srcv)
        pltpu.sync_copy(dst_hbm.at[pl.ds(e0, CH)], dstv)
        pltpu.sync_copy(se_hbm.at[pl.ds(r0, CH // L), :], sebuf)
        for g in range(CH // L):
            dstg = dstv[pl.ds(g * L, L)]
            srcg = srcv[pl.ds(g * L, L)]
            gv = jnp.full((L,), g, jnp.int32)
            for h in range(H):
                fi = dstg * 8 + h
                fj = srcg * 8 + (H + h)
                ce = lanes * 8 + h
                a = (plsc.load_gather(tab, [fi >> 7, fi & 127])
                     + plsc.load_gather(tab, [fj >> 7, fj & 127])
                     + plsc.load_gather(sebuf, [gv, ce]))
                a = jnp.where(a >= 0.0, a, 0.2 * a)
                ex = jnp.exp(a)
                plsc.store_scatter(exbuf, [gv, ce], ex)
                plsc.addupdate_scatter(denl, [dstg * H + h], ex)
        pltpu.sync_copy(exbuf, ex_hbm.at[pl.ds(r0, CH // L), :])
        return jnp.int32(0)

    lax.fori_loop(jnp.int32(0), nch, _chunk, jnp.int32(0))

    @pl.when(c == 0)
    def _():
        pltpu.sync_copy(denl, den0_hbm.at[pl.ds(s * jnp.int32(DENW), DENW)])

    @pl.when(c == 1)
    def _():
        pltpu.sync_copy(denl, den1_hbm.at[pl.ds(s * jnp.int32(DENW), DENW)])


def _stage_sc1(src, dst, sds, se):
    mesh = plsc.VectorSubcoreMesh(core_axis_name="c", subcore_axis_name="s")
    f = pl.kernel(
        _sc1_body,
        mesh=mesh,
        out_type=[
            jax.ShapeDtypeStruct((EPACK, 8 * L), jnp.float32),
            jax.ShapeDtypeStruct((NS * DENW,), jnp.float32),
            jax.ShapeDtypeStruct((NS * DENW,), jnp.float32),
        ],
        compiler_params=pltpu.CompilerParams(needs_layout_passes=False),
        scratch_types=[
            pltpu.VMEM((CH,), jnp.int32),
            pltpu.VMEM((CH,), jnp.int32),
            pltpu.VMEM((N_PAD // L, 8 * L), jnp.float32),
            pltpu.VMEM((CH // L, 8 * L), jnp.float32),
            pltpu.VMEM((CH // L, 8 * L), jnp.float32),
            pltpu.VMEM((DENW,), jnp.float32),
            pltpu.SemaphoreType.DMA,
        ],
    )
    return f(src, dst, sds, se)


def _a2b_body(ea_ref, we_ref, ex_ref, epx0_ref, epx1_ref, epx2_ref):
    ep = jnp.dot(ea_ref[...], we_ref[...], preferred_element_type=jnp.float32)
    exb = ex_ref[...]
    outs = (epx0_ref, epx1_ref, epx2_ref)
    for h in range(H):
        outs[h][...] = ep[:, h * D:(h + 1) * D] * exb[:, h:h + 1]


def _stage_a2b(ea, we, ex):
    grid = E // _BE
    return pl.pallas_call(
        _a2b_body,
        grid=(grid,),
        in_specs=[
            pl.BlockSpec((_BE, DE), lambda i: (i, _Z)),
            pl.BlockSpec((DE, HD), lambda i: (_Z, _Z)),
            pl.BlockSpec((_BE, 8), lambda i: (i, _Z)),
        ],
        out_specs=[
            pl.BlockSpec((_BE, D), lambda i: (i, _Z)),
            pl.BlockSpec((_BE, D), lambda i: (i, _Z)),
            pl.BlockSpec((_BE, D), lambda i: (i, _Z)),
        ],
        out_shape=[
            jax.ShapeDtypeStruct((E, D), jnp.float32),
            jax.ShapeDtypeStruct((E, D), jnp.float32),
            jax.ShapeDtypeStruct((E, D), jnp.float32),
        ],
    )(ea, we, ex)


_NROWS_T = N_PAD // NS
CH2 = 64
NCH2 = E // CH2
_ZCH = _NROWS_T // CH


def _sc2_body(idx_hbm, xp0_hbm, xp1_hbm, xp2_hbm,
              ep0_hbm, ep1_hbm, ep2_hbm, den0_hbm, den1_hbm,
              ag0_hbm, ag1a_hbm, ag1b_hbm, ag2_hbm, inv_hbm,
              idxb0, srcv0, dstv0, xpr0, epr0,
              idxb1, srcv1, dstv1, xpr1, epr1,
              acc, tmp, invbuf,
              agsp, gsem0, ssem0, gsem1, ssem1):
    c = lax.axis_index("c")
    s = lax.axis_index("s")
    lanes = jnp.arange(L, dtype=jnp.int32)

    def _zz(i, _):
        acc[pl.ds(i * jnp.int32(L), L)] = jnp.zeros((L,), jnp.float32)
        return jnp.int32(0)
    lax.fori_loop(jnp.int32(0), jnp.int32(RW // L), _zz, jnp.int32(0))
    for part in (den0_hbm, den1_hbm):
        for t in range(NS):
            pltpu.sync_copy(
                part.at[pl.ds(jnp.int32(t * DENW) + s * jnp.int32(RW), RW)],
                tmp)

            def _red(i, _):
                sl = pl.ds(i * jnp.int32(L), L)
                acc[sl] = acc[sl] + tmp[sl]
                return jnp.int32(0)
            lax.fori_loop(jnp.int32(0), jnp.int32(RW // L), _red, jnp.int32(0))

    def _inv(b, _):
        for h in range(H):
            idx = (b * jnp.int32(L) + lanes) * jnp.int32(H) + h
            v = plsc.load_gather(acc, [idx])
            iv = 1.0 / jnp.maximum(v, 1e-16)
            plsc.store_scatter(invbuf, [jnp.full((L,), 0, jnp.int32) + b,
                                        lanes * 8 + h], iv)
        return jnp.int32(0)
    lax.fori_loop(jnp.int32(0), jnp.int32(RW // (L * H)), _inv, jnp.int32(0))

    @pl.when(c == 0)
    def _():
        pltpu.sync_copy(
            invbuf, inv_hbm.at[pl.ds(s * jnp.int32(RW // (L * H)),
                                     RW // (L * H)), :])

    def _zero_aggr():
        def _zr(r, _):
            for j in range(D // L):
                xpr0[r, pl.ds(j * L, L)] = jnp.zeros((L,), jnp.float32)
            return jnp.int32(0)
        lax.fori_loop(jnp.int32(0), jnp.int32(CH2), _zr, jnp.int32(0))
        for k in range(_NROWS_T // CH2):
            pltpu.sync_copy(
                xpr0,
                agsp.at[pl.ds(s * jnp.int32(_NROWS_T) + jnp.int32(k * CH2),
                              CH2), :])

    def _copy_out(dst_ref):
        for k in range(_ZCH):
            r0 = s * jnp.int32(_NROWS_T) + jnp.int32(k * CH)
            pltpu.sync_copy(agsp.at[pl.ds(r0, CH), :],
                            dst_ref.at[pl.ds(r0, CH), :])

    bufs = ((idxb0, srcv0, dstv0, xpr0, epr0, gsem0, ssem0),
            (idxb1, srcv1, dstv1, xpr1, epr1, gsem1, ssem1))

    def _run(xp_hbm, ep_hbm, ch0, nch):
        def _e0(i):
            return (ch0 + i * jnp.int32(NS)) * jnp.int32(CH2)

        def _fetch(i, b):
            ib, sv, dv, xr, er, gs, ss = bufs[b]
            pltpu.sync_copy(idx_hbm.at[pl.ds((ch0 + i * jnp.int32(NS))
                                             * jnp.int32(2 * CH2),
                                             2 * CH2)], ib)
            for j in range(CH2 // L):
                sv[pl.ds(j * L, L)] = ib[pl.ds(j * L, L)]
                dv[pl.ds(j * L, L)] = ib[pl.ds(CH2 + j * L, L)]
            pltpu.async_copy(xp_hbm.at[sv], xr, gs)
            pltpu.async_copy(ep_hbm.at[pl.ds(_e0(i), CH2), :], er, gs)

        def _drain_scatter(b):
            ib, sv, dv, xr, er, gs, ss = bufs[b]
            pltpu.make_async_copy(er, agsp.at[dv], ss).wait()

        def _process(i, b):
            ib, sv, dv, xr, er, gs, ss = bufs[b]
            pltpu.make_async_copy(xp_hbm.at[sv], xr, gs).wait()
            pltpu.make_async_copy(ep_hbm.at[pl.ds(_e0(i), CH2), :],
                                  er, gs).wait()

            def _row(r2, _2):
                for rr in range(2):
                    r = r2 * jnp.int32(2) + rr
                    for j in range(D // L):
                        sl = pl.ds(j * L, L)
                        er[r, sl] = er[r, sl] * xr[r, sl]
                return jnp.int32(0)
            lax.fori_loop(jnp.int32(0), jnp.int32(CH2 // 2), _row,
                          jnp.int32(0))
            pltpu.async_copy(er, agsp.at[dv], ss, add=True)

        _fetch(jnp.int32(0), 0)
        npair = (nch + jnp.int32(1)) // jnp.int32(2)

        def _pair(k2, _):
            i0 = k2 * jnp.int32(2)
            for b in (0, 1):
                i = i0 + b
                nb = 1 - b

                @pl.when(i < nch)
                def _():
                    @pl.when(i + jnp.int32(1) < nch)
                    def _():
                        @pl.when(i >= jnp.int32(1))
                        def _():
                            _drain_scatter(nb)
                        _fetch(i + jnp.int32(1), nb)
                    _process(i, b)
            return jnp.int32(0)
        lax.fori_loop(jnp.int32(0), npair, _pair, jnp.int32(0))

        _drain_scatter(0)
        _drain_scatter(1)

    _zero_aggr()
    plsc.subcore_barrier()

    ncha = (jnp.int32(NCH2) - s + jnp.int32(NS - 1)) // jnp.int32(NS)

    @pl.when(c == 0)
    def _():
        _run(xp0_hbm, ep0_hbm, s, ncha)

    @pl.when(c == 1)
    def _():
        _run(xp2_hbm, ep2_hbm, s, ncha)

    plsc.subcore_barrier()

    @pl.when(c == 0)
    def _():
        _copy_out(ag0_hbm)

    @pl.when(c == 1)
    def _():
        _copy_out(ag2_hbm)

    plsc.subcore_barrier()
    _zero_aggr()
    plsc.subcore_barrier()

    nchb = (jnp.int32(NCH2 // 2) - s + jnp.int32(NS - 1)) // jnp.int32(NS)

    @pl.when(c == 0)
    def _():
        _run(xp1_hbm, ep1_hbm, s, nchb)

    @pl.when(c == 1)
    def _():
        _run(xp1_hbm, ep1_hbm, jnp.int32(NCH2 // 2) + s, nchb)

    plsc.subcore_barrier()

    @pl.when(c == 0)
    def _():
        _copy_out(ag1a_hbm)

    @pl.when(c == 1)
    def _():
        _copy_out(ag1b_hbm)


def _stage_sc2(idxcat, xp0, xp1, xp2, ep0, ep1, ep2, den0, den1):
    mesh = plsc.VectorSubcoreMesh(core_axis_name="c", subcore_axis_name="s")
    f = pl.kernel(
        _sc2_body,
        mesh=mesh,
        out_type=[
            jax.ShapeDtypeStruct((N_PAD, D), jnp.float32),
            jax.ShapeDtypeStruct((N_PAD, D), jnp.float32),
            jax.ShapeDtypeStruct((N_PAD, D), jnp.float32),
            jax.ShapeDtypeStruct((N_PAD, D), jnp.float32),
            jax.ShapeDtypeStruct((N_PAD // L, 8 * L), jnp.float32),
        ],
        compiler_params=pltpu.CompilerParams(needs_layout_passes=False),
        scratch_types=[
            pltpu.VMEM((2 * CH2,), jnp.int32),
            pltpu.VMEM((CH2,), jnp.int32),
            pltpu.VMEM((CH2,), jnp.int32),
            pltpu.VMEM((CH2, D), jnp.float32),
            pltpu.VMEM((CH2, D), jnp.float32),
            pltpu.VMEM((2 * CH2,), jnp.int32),
            pltpu.VMEM((CH2,), jnp.int32),
            pltpu.VMEM((CH2,), jnp.int32),
            pltpu.VMEM((CH2, D), jnp.float32),
            pltpu.VMEM((CH2, D), jnp.float32),
            pltpu.VMEM((RW,), jnp.float32),
            pltpu.VMEM((RW,), jnp.float32),
            pltpu.VMEM((RW // (L * H), 8 * L), jnp.float32),
            pltpu.VMEM_SHARED((N_PAD, D), jnp.float32),
            pltpu.SemaphoreType.DMA,
            pltpu.SemaphoreType.DMA,
            pltpu.SemaphoreType.DMA,
            pltpu.SemaphoreType.DMA,
        ],
    )
    return f(idxcat, xp0, xp1, xp2, ep0, ep1, ep2, den0, den1)


_BNF = 1024
_BDF = _BNF * H // (8 * L)


def _final_body(a0_ref, a1a_ref, a1b_ref, a2_ref, inv_ref,
                ws_ref, b_ref, out_ref):
    inv = inv_ref[...]
    heads = (a0_ref[...], a1a_ref[...] + a1b_ref[...], a2_ref[...])
    acc = b_ref[...]
    for h in range(H):
        scaled = heads[h] * inv[:, h:h + 1]
        acc = acc + jnp.dot(scaled, ws_ref[h * D:(h + 1) * D, :],
                            preferred_element_type=jnp.float32)
    out_ref[...] = acc


def _stage_final(a0, a1a, a1b, a2, inv, ws, bias2d):
    grid = N_PAD // _BNF
    return pl.pallas_call(
        _final_body,
        grid=(grid,),
        in_specs=[
            pl.BlockSpec((_BNF, D), lambda i: (i, _Z)),
            pl.BlockSpec((_BNF, D), lambda i: (i, _Z)),
            pl.BlockSpec((_BNF, D), lambda i: (i, _Z)),
            pl.BlockSpec((_BNF, D), lambda i: (i, _Z)),
            pl.BlockSpec((_BNF, 8), lambda i: (i, _Z)),
            pl.BlockSpec((HD, D), lambda i: (_Z, _Z)),
            pl.BlockSpec((1, D), lambda i: (_Z, _Z)),
        ],
        out_specs=[pl.BlockSpec((_BNF, D), lambda i: (i, _Z))],
        out_shape=[jax.ShapeDtypeStruct((N_PAD, D), jnp.float32)],
    )(a0, a1a, a1b, a2, inv, ws, bias2d)[0]


def kernel(x, edge_index, edge_attr, weight_node, weight_edge,
           weight_triplet_att, weight_scale, bias):
    src = edge_index[0].astype(jnp.int32)
    dst = edge_index[1].astype(jnp.int32)
    x_pad = jnp.concatenate(
        [x.astype(jnp.float32),
         jnp.zeros((N_PAD - N, D), jnp.float32)], axis=0)
    ea = edge_attr.astype(jnp.float32)

    xp0, xp1, xp2, sds = _stage_a1(x_pad, weight_node, weight_triplet_att)
    se = _stage_a2a(ea, weight_edge, weight_triplet_att)
    ex, den0, den1 = _stage_sc1(src, dst,
                                sds.reshape(N_PAD // L, 8 * L),
                                se.reshape(EPACK, 8 * L))
    ep0, ep1, ep2 = _stage_a2b(ea, weight_edge, ex.reshape(E, 8))
    idxcat = jnp.concatenate([src.reshape(NCH2, CH2),
                              dst.reshape(NCH2, CH2)],
                             axis=1).reshape(2 * E)
    a0, a1a, a1b, a2, invden = _stage_sc2(idxcat, xp0, xp1, xp2,
                                          ep0, ep1, ep2, den0, den1)
    out = _stage_final(a0, a1a, a1b, a2, invden.reshape(N_PAD, 8),
                       weight_scale,
                       bias.reshape(1, D).astype(jnp.float32))
    return out[:N]
